# baseline (speedup 1.0000x reference)
import functools

import jax
import jax.numpy as jnp
from jax import lax
from jax.experimental import pallas as pl
from jax.experimental.pallas import tpu as pltpu
from jax.experimental.pallas import tpu_sc as plsc

NC = 2
NS = 16
NW = NC * NS
LANES = 16
CHUNK = 128


def _precompute_tables(h, W1a, W1b):
    n, d = h.shape
    blk = 1000

    def body(h_ref, wa_ref, wb_ref, ha_ref, hb_ref):
        hb = h_ref[...]
        ha_ref[...] = jnp.dot(hb, wa_ref[...],
                              preferred_element_type=jnp.float32)
        hb_ref[...] = jnp.dot(hb, wb_ref[...],
                              preferred_element_type=jnp.float32)

    return pl.pallas_call(
        body,
        grid=(n // blk,),
        in_specs=[
            pl.BlockSpec((blk, d), lambda i: (i, 0)),
            pl.BlockSpec((d, d), lambda i: (0, 0)),
            pl.BlockSpec((d, d), lambda i: (0, 0)),
        ],
        out_specs=[
            pl.BlockSpec((blk, d), lambda i: (i, 0)),
            pl.BlockSpec((blk, d), lambda i: (i, 0)),
        ],
        out_shape=[
            jax.ShapeDtypeStruct((n, d), jnp.float32),
            jax.ShapeDtypeStruct((n, d), jnp.float32),
        ],
    )(h, W1a, W1b)


def _sc_gather_add(ha, hb, ei, ej, e_pad, d, q0_frac):
    tch = e_pad // CHUNK
    q0 = int(round(tch * q0_frac / (NS * 2))) * 2
    q1 = tch // NS - q0
    epw0 = q0 * CHUNK
    epw1 = q1 * CHUNK
    epw_max = max(epw0, epw1)

    mesh = plsc.VectorSubcoreMesh(core_axis_name="c", subcore_axis_name="s",
                                  num_cores=NC, num_subcores=NS)

    @functools.partial(
        pl.kernel,
        out_type=jax.ShapeDtypeStruct((e_pad, d), jnp.float32),
        mesh=mesh,
        scratch_types=[
            pltpu.VMEM((epw_max,), jnp.int32),
            pltpu.VMEM((epw_max,), jnp.int32),
            pltpu.VMEM((CHUNK, d), jnp.float32),
            pltpu.VMEM((CHUNK, d), jnp.float32),
            pltpu.VMEM((CHUNK, d), jnp.float32),
            pltpu.VMEM((CHUNK, d), jnp.float32),
            pltpu.VMEM((CHUNK, d), jnp.float32),
            pltpu.VMEM((CHUNK, d), jnp.float32),
            pltpu.SemaphoreType.DMA,
            pltpu.SemaphoreType.DMA,
            pltpu.SemaphoreType.DMA,
            pltpu.SemaphoreType.DMA,
            pltpu.SemaphoreType.DMA,
            pltpu.SemaphoreType.DMA,
        ],
    )
    def gather_kernel(ha_hbm, hb_hbm, ei_hbm, ej_hbm, g_hbm,
                      idx_i, idx_j, a0, b0, o0, a1, b1, o1,
                      sa0, sb0, sa1, sb1, st0, st1):
        cid = lax.axis_index("c")
        sid = lax.axis_index("s")
        base = cid * (NS * epw0) + sid * jnp.where(cid == 0, epw0, epw1)
        epw = jnp.where(cid == 0, epw0, epw1)
        nch = jnp.where(cid == 0, q0, q1)
        bufs = ((a0, b0, o0), (a1, b1, o1))
        sems = ((sa0, sb0), (sa1, sb1))
        stsems = (st0, st1)
        nseg = d // LANES

        pbase = jnp.minimum(base, e_pad - epw_max)
        delta = base - pbase
        pltpu.sync_copy(ei_hbm.at[pl.ds(pbase, epw_max)], idx_i)
        pltpu.sync_copy(ej_hbm.at[pl.ds(pbase, epw_max)], idx_j)

        def issue(k, s):
            loc = delta + k * CHUNK
            pltpu.async_copy(ha_hbm.at[idx_i.at[pl.ds(loc, CHUNK)]],
                             bufs[s][0], sems[s][0])
            pltpu.async_copy(hb_hbm.at[idx_j.at[pl.ds(loc, CHUNK)]],
                             bufs[s][1], sems[s][1])

        def wait_store(k, s):
            pltpu.make_async_copy(bufs[s][2],
                                  g_hbm.at[pl.ds(base + k * CHUNK, CHUNK)],
                                  stsems[s]).wait()

        def process(k, s, first):
            loc = delta + k * CHUNK
            pltpu.make_async_copy(ha_hbm.at[idx_i.at[pl.ds(loc, CHUNK)]],
                                  bufs[s][0], sems[s][0]).wait()
            pltpu.make_async_copy(hb_hbm.at[idx_j.at[pl.ds(loc, CHUNK)]],
                                  bufs[s][1], sems[s][1]).wait()
            if not first:
                wait_store(k, s)

            def row_body(r, c2):
                for seg in range(nseg):
                    sl = pl.ds(seg * LANES, LANES)
                    bufs[s][2][r, sl] = bufs[s][0][r, sl] + bufs[s][1][r, sl]
                return c2

            lax.fori_loop(0, CHUNK, row_body, 0, unroll=4)
            pltpu.async_copy(bufs[s][2],
                             g_hbm.at[pl.ds(base + k * CHUNK, CHUNK)],
                             stsems[s])

        issue(0, 0)
        issue(1, 1)
        process(0, 0, True)
        issue(2, 0)
        process(1, 1, True)
        issue(3, 1)

        def body(i, carry):
            k0 = 2 * i + 2
            process(k0, 0, False)
            issue(k0 + 2, 0)
            process(k0 + 1, 1, False)
            issue(k0 + 3, 1)
            return carry

        lax.fori_loop(0, nch // 2 - 2, body, 0)
        process(nch - 2, 0, False)
        process(nch - 1, 1, False)
        wait_store(nch - 2, 0)
        wait_store(nch - 1, 1)

    return gather_kernel(ha, hb, ei, ej)


def _edge_mlp(g, d2, w1c, b1, gamma, beta, W2, b2, W3, b3, w4, b4,
              e_pad, d, blk0, blkmax):
    blk = 1280

    def silu(v):
        return v * (1.0 / (1.0 + jnp.exp(-v)))

    def body(g_ref, d2_ref, w1c_ref, b1_ref, gam_ref, bet_ref,
             w2_ref, b2_ref, w3_ref, b3_ref, w4_ref, b4_ref, t_ref):
        gv = g_ref[...]
        pre = gv + d2_ref[...] * w1c_ref[...] + b1_ref[...]
        u = silu(pre)
        mu = jnp.mean(u, axis=-1, keepdims=True)
        var = jnp.mean((u - mu) ** 2, axis=-1, keepdims=True)
        v = (u - mu) * lax.rsqrt(var + 1e-5) * gam_ref[...] + bet_ref[...]
        a = silu(jnp.dot(v.astype(jnp.bfloat16), w2_ref[...],
                         preferred_element_type=jnp.float32) + b2_ref[...])
        m3 = silu(jnp.dot(a.astype(jnp.bfloat16), w3_ref[...],
                          preferred_element_type=jnp.float32) + b3_ref[...])
        t = lax.dot_general(w4_ref[...], m3, (((1,), (1,)), ((), ())),
                            preferred_element_type=jnp.float32)
        t_ref[...] = t + b4_ref[...]

    wspec = pl.BlockSpec((d, d), lambda i: (0, 0))
    rowspec = pl.BlockSpec((1, d), lambda i: (0, 0))
    return pl.pallas_call(
        body,
        grid=(e_pad // blk,),
        in_specs=[
            pl.BlockSpec((blk, d), lambda i: (i, 0)),
            pl.BlockSpec((blk, 1), lambda i: (jnp.minimum(blk0 + i, blkmax),
                                              0)),
            rowspec, rowspec, rowspec, rowspec,
            wspec, rowspec, wspec, rowspec, rowspec,
            pl.BlockSpec((1, 1), lambda i: (0, 0)),
        ],
        out_specs=pl.BlockSpec((1, blk), lambda i: (0, i)),
        out_shape=jax.ShapeDtypeStruct((1, e_pad), jnp.float32),
    )(g, d2, w1c, b1, gamma, beta, W2, b2, W3, b3, w4, b4)


def _sc_scatter(ei, t, dxt, e_pad, n):
    epw = e_pad // NW
    nch = epw // CHUNK
    n3 = 3 * n
    n3p = ((n3 + NS * LANES - 1) // (NS * LANES)) * (NS * LANES)
    tile_words = n3p // NS

    mesh = plsc.VectorSubcoreMesh(core_axis_name="c", subcore_axis_name="s",
                                  num_cores=NC, num_subcores=NS)

    @functools.partial(
        pl.kernel,
        out_type=jax.ShapeDtypeStruct((NC * n3p,), jnp.float32),
        mesh=mesh,
        compiler_params=pltpu.CompilerParams(needs_layout_passes=False),
        scratch_types=[
            pltpu.VMEM((CHUNK,), jnp.int32),
            pltpu.VMEM((CHUNK,), jnp.float32),
            pltpu.VMEM((3 * CHUNK,), jnp.float32),
            pltpu.VMEM((CHUNK,), jnp.int32),
            pltpu.VMEM((CHUNK,), jnp.float32),
            pltpu.VMEM((tile_words,), jnp.float32),
            pltpu.VMEM_SHARED((n3p,), jnp.float32),
        ],
    )
    def scatter_kernel(ei_hbm, t_hbm, dxf_hbm, out_hbm,
                       ei_buf, t_buf, dx_buf, idx_buf, val_buf, stage, acc):
        cid = lax.axis_index("c")
        sid = lax.axis_index("s")
        wid = sid * NC + cid
        base = wid * epw

        def zero_body(j, carry):
            stage[pl.ds(j * LANES, LANES)] = jnp.zeros((LANES,), jnp.float32)
            return carry

        lax.fori_loop(0, tile_words // LANES, zero_body, 0)
        pltpu.sync_copy(stage, acc.at[pl.ds(sid * tile_words, tile_words)])
        plsc.subcore_barrier()

        lane3 = lax.iota(jnp.int32, LANES) * 3

        def chunk_body(k, carry):
            off = base + k * CHUNK
            pltpu.sync_copy(ei_hbm.at[pl.ds(off, CHUNK)], ei_buf)
            pltpu.sync_copy(t_hbm.at[pl.ds(off, CHUNK)], t_buf)
            pltpu.sync_copy(dxf_hbm.at[pl.ds(3 * off, 3 * CHUNK)], dx_buf)
            for c in range(3):

                def seg_body(j, c2):
                    sl = pl.ds(j * LANES, LANES)
                    idx_buf[sl] = ei_buf[sl] * 3 + c
                    dv = plsc.load_gather(dx_buf,
                                          [lane3 + (3 * LANES * j + c)])
                    val_buf[sl] = dv * t_buf[sl]
                    return c2

                lax.fori_loop(0, CHUNK // LANES, seg_body, 0, unroll=True)
                pltpu.sync_copy(val_buf, acc.at[idx_buf], add=True)
            return carry

        lax.fori_loop(0, nch, chunk_body, 0)
        plsc.subcore_barrier()
        sl = pl.ds(sid * tile_words, tile_words)
        pltpu.sync_copy(acc.at[sl], stage)
        pltpu.sync_copy(stage,
                        out_hbm.at[pl.ds(cid * n3p + sid * tile_words,
                                         tile_words)])

    return scatter_kernel(ei, t, dxt)


def kernel(h, x, e, dx, d2, W1, b1, gamma, beta, W2, b2, W3, b3, W4, b4):
    n, d = h.shape
    e_cnt = e.shape[1]
    quantum = NW * CHUNK * 2
    e_pad = ((e_cnt + quantum - 1) // quantum) * quantum
    pad = e_pad - e_cnt

    ei = e[0].astype(jnp.int32)
    ej = e[1].astype(jnp.int32)
    ei_p = jnp.pad(ei, (0, pad))
    ej_p = jnp.pad(ej, (0, pad))
    dxf_p = jnp.pad(dx.reshape(3 * e_cnt), (0, 3 * pad))

    W1a = W1[:d]
    W1b = W1[d:2 * d]
    w1c = W1[2 * d].reshape(1, d)

    ha, hb = _precompute_tables(h, W1a, W1b)
    nslice = 8
    assert e_pad % (nslice * quantum) == 0 or nslice == 1
    slen = e_pad // nslice
    ts = []
    mlp_blk = 1280
    blkmax = e_cnt // mlp_blk - 1
    for si in range(nslice):
        sl = slice(si * slen, (si + 1) * slen)
        g = _sc_gather_add(ha, hb, ei_p[sl], ej_p[sl], slen, d,
                           q0_frac=0.5)
        ts.append(_edge_mlp(g, d2, w1c, b1.reshape(1, d),
                            gamma.reshape(1, d), beta.reshape(1, d),
                            W2.astype(jnp.bfloat16), b2.reshape(1, d),
                            W3.astype(jnp.bfloat16), b3.reshape(1, d),
                            W4.reshape(1, d), b4.reshape(1, 1), slen, d,
                            si * (slen // mlp_blk), blkmax))
    t = jnp.concatenate(ts, axis=1)
    parts = _sc_scatter(ei_p, t.reshape(e_pad), dxf_p, e_pad, n)
    n3p = parts.shape[0] // NC
    agg = (parts[:n3p] + parts[n3p:])[:3 * n].reshape(n, 3)
    return x + agg


# speedup vs baseline: 1.0133x; 1.0133x over previous
import functools

import jax
import jax.numpy as jnp
from jax import lax
from jax.experimental import pallas as pl
from jax.experimental.pallas import tpu as pltpu
from jax.experimental.pallas import tpu_sc as plsc

NC = 2
NS = 16
NW = NC * NS
LANES = 16
CHUNK = 128


def _precompute_tables(h, W1a, W1b):
    n, d = h.shape
    blk = 1000

    def body(h_ref, wa_ref, wb_ref, ha_ref, hb_ref):
        hb = h_ref[...]
        ha_ref[...] = jnp.dot(hb, wa_ref[...],
                              preferred_element_type=jnp.float32)
        hb_ref[...] = jnp.dot(hb, wb_ref[...],
                              preferred_element_type=jnp.float32)

    return pl.pallas_call(
        body,
        grid=(n // blk,),
        in_specs=[
            pl.BlockSpec((blk, d), lambda i: (i, 0)),
            pl.BlockSpec((d, d), lambda i: (0, 0)),
            pl.BlockSpec((d, d), lambda i: (0, 0)),
        ],
        out_specs=[
            pl.BlockSpec((blk, d), lambda i: (i, 0)),
            pl.BlockSpec((blk, d), lambda i: (i, 0)),
        ],
        out_shape=[
            jax.ShapeDtypeStruct((n, d), jnp.float32),
            jax.ShapeDtypeStruct((n, d), jnp.float32),
        ],
    )(h, W1a, W1b)


def _sc_gather_add(ha, hb, ei, ej, e_pad, d, q0_frac):
    tch = e_pad // CHUNK
    q0 = int(round(tch * q0_frac / (NS * 2))) * 2
    q1 = tch // NS - q0
    epw0 = q0 * CHUNK
    epw1 = q1 * CHUNK
    epw_max = max(epw0, epw1)

    mesh = plsc.VectorSubcoreMesh(core_axis_name="c", subcore_axis_name="s",
                                  num_cores=NC, num_subcores=NS)

    @functools.partial(
        pl.kernel,
        out_type=jax.ShapeDtypeStruct((e_pad, d), jnp.float32),
        mesh=mesh,
        scratch_types=[
            pltpu.VMEM((epw_max,), jnp.int32),
            pltpu.VMEM((epw_max,), jnp.int32),
            pltpu.VMEM((CHUNK, d), jnp.float32),
            pltpu.VMEM((CHUNK, d), jnp.float32),
            pltpu.VMEM((CHUNK, d), jnp.float32),
            pltpu.VMEM((CHUNK, d), jnp.float32),
            pltpu.VMEM((CHUNK, d), jnp.float32),
            pltpu.VMEM((CHUNK, d), jnp.float32),
            pltpu.SemaphoreType.DMA,
            pltpu.SemaphoreType.DMA,
            pltpu.SemaphoreType.DMA,
            pltpu.SemaphoreType.DMA,
            pltpu.SemaphoreType.DMA,
            pltpu.SemaphoreType.DMA,
        ],
    )
    def gather_kernel(ha_hbm, hb_hbm, ei_hbm, ej_hbm, g_hbm,
                      idx_i, idx_j, a0, b0, o0, a1, b1, o1,
                      sa0, sb0, sa1, sb1, st0, st1):
        cid = lax.axis_index("c")
        sid = lax.axis_index("s")
        base = cid * (NS * epw0) + sid * jnp.where(cid == 0, epw0, epw1)
        epw = jnp.where(cid == 0, epw0, epw1)
        nch = jnp.where(cid == 0, q0, q1)
        bufs = ((a0, b0, o0), (a1, b1, o1))
        sems = ((sa0, sb0), (sa1, sb1))
        stsems = (st0, st1)
        nseg = d // LANES

        pbase = jnp.minimum(base, e_pad - epw_max)
        delta = base - pbase
        pltpu.sync_copy(ei_hbm.at[pl.ds(pbase, epw_max)], idx_i)
        pltpu.sync_copy(ej_hbm.at[pl.ds(pbase, epw_max)], idx_j)

        def issue(k, s):
            loc = delta + k * CHUNK
            pltpu.async_copy(ha_hbm.at[idx_i.at[pl.ds(loc, CHUNK)]],
                             bufs[s][0], sems[s][0])
            pltpu.async_copy(hb_hbm.at[idx_j.at[pl.ds(loc, CHUNK)]],
                             bufs[s][1], sems[s][1])

        def wait_store(k, s):
            pltpu.make_async_copy(bufs[s][2],
                                  g_hbm.at[pl.ds(base + k * CHUNK, CHUNK)],
                                  stsems[s]).wait()

        def process(k, s, first):
            loc = delta + k * CHUNK
            pltpu.make_async_copy(ha_hbm.at[idx_i.at[pl.ds(loc, CHUNK)]],
                                  bufs[s][0], sems[s][0]).wait()
            pltpu.make_async_copy(hb_hbm.at[idx_j.at[pl.ds(loc, CHUNK)]],
                                  bufs[s][1], sems[s][1]).wait()
            if not first:
                wait_store(k, s)

            def row_body(r, c2):
                for seg in range(nseg):
                    sl = pl.ds(seg * LANES, LANES)
                    bufs[s][2][r, sl] = bufs[s][0][r, sl] + bufs[s][1][r, sl]
                return c2

            lax.fori_loop(0, CHUNK, row_body, 0, unroll=4)
            pltpu.async_copy(bufs[s][2],
                             g_hbm.at[pl.ds(base + k * CHUNK, CHUNK)],
                             stsems[s])

        issue(0, 0)
        issue(1, 1)
        process(0, 0, True)
        issue(2, 0)
        process(1, 1, True)
        issue(3, 1)

        def body(i, carry):
            k0 = 2 * i + 2
            process(k0, 0, False)
            issue(k0 + 2, 0)
            process(k0 + 1, 1, False)
            issue(k0 + 3, 1)
            return carry

        lax.fori_loop(0, nch // 2 - 2, body, 0)
        process(nch - 2, 0, False)
        process(nch - 1, 1, False)
        wait_store(nch - 2, 0)
        wait_store(nch - 1, 1)

    return gather_kernel(ha, hb, ei, ej)


def _edge_mlp(g, d2, w1c, b1, gamma, beta, W2, b2, W3, b3, w4, b4,
              e_pad, d, blk0, blkmax):
    blk = 1280

    def silu(v):
        return v * (1.0 / (1.0 + jnp.exp(-v)))

    def body(g_ref, d2_ref, w1c_ref, b1_ref, gam_ref, bet_ref,
             w2_ref, b2_ref, w3_ref, b3_ref, w4_ref, b4_ref, t_ref):
        gv = g_ref[...]
        pre = gv + d2_ref[...] * w1c_ref[...] + b1_ref[...]
        u = silu(pre)
        mu = jnp.mean(u, axis=-1, keepdims=True)
        var = jnp.mean((u - mu) ** 2, axis=-1, keepdims=True)
        v = (u - mu) * lax.rsqrt(var + 1e-5) * gam_ref[...] + bet_ref[...]
        a = silu(jnp.dot(v.astype(jnp.bfloat16), w2_ref[...],
                         preferred_element_type=jnp.float32) + b2_ref[...])
        m3 = silu(jnp.dot(a.astype(jnp.bfloat16), w3_ref[...],
                          preferred_element_type=jnp.float32) + b3_ref[...])
        t = lax.dot_general(w4_ref[...], m3, (((1,), (1,)), ((), ())),
                            preferred_element_type=jnp.float32)
        t_ref[...] = t + b4_ref[...]

    wspec = pl.BlockSpec((d, d), lambda i: (0, 0))
    rowspec = pl.BlockSpec((1, d), lambda i: (0, 0))
    return pl.pallas_call(
        body,
        grid=(e_pad // blk,),
        in_specs=[
            pl.BlockSpec((blk, d), lambda i: (i, 0)),
            pl.BlockSpec((blk, 1), lambda i: (jnp.minimum(blk0 + i, blkmax),
                                              0)),
            rowspec, rowspec, rowspec, rowspec,
            wspec, rowspec, wspec, rowspec, rowspec,
            pl.BlockSpec((1, 1), lambda i: (0, 0)),
        ],
        out_specs=pl.BlockSpec((1, blk), lambda i: (0, i)),
        out_shape=jax.ShapeDtypeStruct((1, e_pad), jnp.float32),
    )(g, d2, w1c, b1, gamma, beta, W2, b2, W3, b3, w4, b4)


def _sc_scatter(ei, t, dxt, e_pad, n):
    epw = e_pad // NW
    nch = epw // CHUNK
    n3 = 3 * n
    n3p = ((n3 + NS * LANES - 1) // (NS * LANES)) * (NS * LANES)
    tile_words = n3p // NS

    mesh = plsc.VectorSubcoreMesh(core_axis_name="c", subcore_axis_name="s",
                                  num_cores=NC, num_subcores=NS)

    @functools.partial(
        pl.kernel,
        out_type=jax.ShapeDtypeStruct((NC * n3p,), jnp.float32),
        mesh=mesh,
        compiler_params=pltpu.CompilerParams(needs_layout_passes=False),
        scratch_types=[
            pltpu.VMEM((CHUNK,), jnp.int32),
            pltpu.VMEM((CHUNK,), jnp.float32),
            pltpu.VMEM((3 * CHUNK,), jnp.float32),
            pltpu.VMEM((CHUNK,), jnp.int32),
            pltpu.VMEM((CHUNK,), jnp.float32),
            pltpu.VMEM((tile_words,), jnp.float32),
            pltpu.VMEM_SHARED((n3p,), jnp.float32),
        ],
    )
    def scatter_kernel(ei_hbm, t_hbm, dxf_hbm, out_hbm,
                       ei_buf, t_buf, dx_buf, idx_buf, val_buf, stage, acc):
        cid = lax.axis_index("c")
        sid = lax.axis_index("s")
        wid = sid * NC + cid
        base = wid * epw

        def zero_body(j, carry):
            stage[pl.ds(j * LANES, LANES)] = jnp.zeros((LANES,), jnp.float32)
            return carry

        lax.fori_loop(0, tile_words // LANES, zero_body, 0)
        pltpu.sync_copy(stage, acc.at[pl.ds(sid * tile_words, tile_words)])
        plsc.subcore_barrier()

        lane3 = lax.iota(jnp.int32, LANES) * 3

        def chunk_body(k, carry):
            off = base + k * CHUNK
            pltpu.sync_copy(ei_hbm.at[pl.ds(off, CHUNK)], ei_buf)
            pltpu.sync_copy(t_hbm.at[pl.ds(off, CHUNK)], t_buf)
            pltpu.sync_copy(dxf_hbm.at[pl.ds(3 * off, 3 * CHUNK)], dx_buf)
            for c in range(3):

                def seg_body(j, c2):
                    sl = pl.ds(j * LANES, LANES)
                    idx_buf[sl] = ei_buf[sl] * 3 + c
                    dv = plsc.load_gather(dx_buf,
                                          [lane3 + (3 * LANES * j + c)])
                    val_buf[sl] = dv * t_buf[sl]
                    return c2

                lax.fori_loop(0, CHUNK // LANES, seg_body, 0, unroll=True)
                pltpu.sync_copy(val_buf, acc.at[idx_buf], add=True)
            return carry

        lax.fori_loop(0, nch, chunk_body, 0)
        plsc.subcore_barrier()
        sl = pl.ds(sid * tile_words, tile_words)
        pltpu.sync_copy(acc.at[sl], stage)
        pltpu.sync_copy(stage,
                        out_hbm.at[pl.ds(cid * n3p + sid * tile_words,
                                         tile_words)])

    return scatter_kernel(ei, t, dxt)


def kernel(h, x, e, dx, d2, W1, b1, gamma, beta, W2, b2, W3, b3, W4, b4):
    n, d = h.shape
    e_cnt = e.shape[1]
    quantum = NW * CHUNK * 2
    e_pad = ((e_cnt + quantum - 1) // quantum) * quantum
    pad = e_pad - e_cnt

    ei = e[0].astype(jnp.int32)
    ej = e[1].astype(jnp.int32)
    ei_p = jnp.pad(ei, (0, pad))
    ej_p = jnp.pad(ej, (0, pad))
    dxf_p = jnp.pad(dx.reshape(3 * e_cnt), (0, 3 * pad))

    W1a = W1[:d]
    W1b = W1[d:2 * d]
    w1c = W1[2 * d].reshape(1, d)

    ha, hb = _precompute_tables(h, W1a, W1b)
    nslice = 4
    assert e_pad % (nslice * quantum) == 0 or nslice == 1
    slen = e_pad // nslice
    ts = []
    mlp_blk = 1280
    blkmax = e_cnt // mlp_blk - 1
    for si in range(nslice):
        sl = slice(si * slen, (si + 1) * slen)
        g = _sc_gather_add(ha, hb, ei_p[sl], ej_p[sl], slen, d,
                           q0_frac=0.5)
        ts.append(_edge_mlp(g, d2, w1c, b1.reshape(1, d),
                            gamma.reshape(1, d), beta.reshape(1, d),
                            W2.astype(jnp.bfloat16), b2.reshape(1, d),
                            W3.astype(jnp.bfloat16), b3.reshape(1, d),
                            W4.reshape(1, d), b4.reshape(1, 1), slen, d,
                            si * (slen // mlp_blk), blkmax))
    t = jnp.concatenate(ts, axis=1)
    parts = _sc_scatter(ei_p, t.reshape(e_pad), dxf_p, e_pad, n)
    n3p = parts.shape[0] // NC
    agg = (parts[:n3p] + parts[n3p:])[:3 * n].reshape(n, 3)
    return x + agg
